# baseline (speedup 1.0000x reference)
import functools

import jax
import jax.numpy as jnp
from jax import lax
from jax.experimental import pallas as pl
from jax.experimental.pallas import tpu as pltpu
from jax.experimental.pallas import tpu_sc as plsc

_N, _E, _S, _G = 10240, 40960, 640, 32
_NF, _EA, _H = 11, 5, 64
_HP = 128
_KH = 128
_HH = _H * _H
_ZR = 1000
_NT = 5
_NC, _NS = 2, 16
_NW = _NC * _NS
_EB = 256
_NB = 256
_TB = 512
_GCH = 128
_f32 = jnp.float32

_sc_mesh = plsc.VectorSubcoreMesh(
    core_axis_name="c", subcore_axis_name="s", num_cores=_NC, num_subcores=_NS)


def _emb_body(z_ref, nt_ref, zt_ref, ntt_ref, out_ref):
    z = z_ref[0, 0, :]
    nt = nt_ref[0, 0, :]
    ohz = (z[:, None] == lax.broadcasted_iota(jnp.int32, (_NB, _ZR), 1)).astype(_f32)
    ohn = (nt[:, None] == lax.broadcasted_iota(jnp.int32, (_NB, _NT), 1)).astype(_f32)
    out_ref[...] = (jnp.dot(ohz, zt_ref[...], preferred_element_type=_f32)
                    + jnp.dot(ohn, ntt_ref[...], preferred_element_type=_f32))


def _embed(z, node_type, ztp, nttp):
    nb = _N // _NB
    z3 = z.reshape(nb, 1, _NB)
    nt3 = node_type.reshape(nb, 1, _NB)
    return pl.pallas_call(
        _emb_body,
        grid=(nb,),
        in_specs=[
            pl.BlockSpec((1, 1, _NB), lambda i: (i, 0, 0)),
            pl.BlockSpec((1, 1, _NB), lambda i: (i, 0, 0)),
            pl.BlockSpec((_ZR, _HP), lambda i: (0, 0)),
            pl.BlockSpec((_NT, _HP), lambda i: (0, 0)),
        ],
        out_specs=pl.BlockSpec((_NB, _HP), lambda i: (i, 0)),
        out_shape=jax.ShapeDtypeStruct((_N, _HP), _f32),
    )(z3, nt3, ztp, nttp)


def _gather_rows(table, idx):
    per = _E // _NW
    nch = per // _GCH

    @functools.partial(
        pl.kernel,
        out_type=jax.ShapeDtypeStruct((_E, _HP), _f32),
        mesh=_sc_mesh,
        scratch_types=[
            pltpu.VMEM((_GCH,), jnp.int32),
            pltpu.VMEM((_GCH, _HP), _f32),
            pltpu.SemaphoreType.DMA,
        ],
    )
    def k(tab_hbm, idx_hbm, out_hbm, idx_v, rows_v, sem):
        wid = lax.axis_index("s") * _NC + lax.axis_index("c")
        base = wid * per
        for c in range(nch):
            off = base + c * _GCH
            pltpu.sync_copy(idx_hbm.at[pl.ds(off, _GCH)], idx_v)
            pltpu.async_copy(tab_hbm.at[idx_v], rows_v, sem).wait()
            pltpu.sync_copy(rows_v, out_hbm.at[pl.ds(off, _GCH)])

    return k(table, idx)


def _scatter_add(msg, dst, zinit):
    per = _E // _NW
    nch = per // _GCH
    rps = _N // _NS

    @functools.partial(
        pl.kernel,
        out_type=jax.ShapeDtypeStruct((_NC * _N, _HP), _f32),
        mesh=_sc_mesh,
        scratch_types=[
            pltpu.VMEM((_GCH,), jnp.int32),
            pltpu.VMEM((_GCH, _HP), _f32),
            pltpu.VMEM_SHARED((_N, _HP), _f32),
        ],
    )
    def k(msg_hbm, dst_hbm, z_hbm, out_hbm, idx_v, rows_v, agg_sh):
        cid = lax.axis_index("c")
        sid = lax.axis_index("s")
        wid = sid * _NC + cid
        pltpu.sync_copy(z_hbm, agg_sh.at[pl.ds(sid * rps, rps)])
        plsc.subcore_barrier()
        base = wid * per
        for c in range(nch):
            off = base + c * _GCH
            pltpu.sync_copy(dst_hbm.at[pl.ds(off, _GCH)], idx_v)
            pltpu.sync_copy(msg_hbm.at[pl.ds(off, _GCH)], rows_v)
            pltpu.sync_copy(rows_v, agg_sh.at[idx_v], add=True)
        plsc.subcore_barrier()
        pltpu.sync_copy(agg_sh.at[pl.ds(sid * rps, rps)],
                        out_hbm.at[pl.ds(cid * _N + sid * rps, rps)])

    return k(msg, dst, zinit)


def _dense_body(ea_ref, hs_ref, w1_ref, b1_ref, w2_ref, b2_ref, out_ref):
    f = jnp.maximum(
        jnp.dot(ea_ref[...], w1_ref[...], preferred_element_type=_f32)
        + b1_ref[...], 0.0)
    q = jnp.dot(f, w2_ref[...], preferred_element_type=_f32) + b2_ref[...]
    hs = hs_ref[...][:, :_H]
    q3 = q.reshape(_EB, _H, _H)
    msg = jnp.sum(q3 * hs[:, :, None], axis=1)
    out_ref[...] = jnp.concatenate(
        [msg, jnp.zeros((_EB, _HP - _H), _f32)], axis=1)


def _dense(ea, hs, w1, b1r, w2, b2r):
    nb = _E // _EB
    return pl.pallas_call(
        _dense_body,
        grid=(nb,),
        in_specs=[
            pl.BlockSpec((_EB, _EA), lambda i: (i, 0)),
            pl.BlockSpec((_EB, _HP), lambda i: (i, 0)),
            pl.BlockSpec((_EA, _KH), lambda i: (0, 0)),
            pl.BlockSpec((1, _KH), lambda i: (0, 0)),
            pl.BlockSpec((_KH, _HH), lambda i: (0, 0)),
            pl.BlockSpec((1, _HH), lambda i: (0, 0)),
        ],
        out_specs=pl.BlockSpec((_EB, _HP), lambda i: (i, 0)),
        out_shape=jax.ShapeDtypeStruct((_E, _HP), _f32),
    )(ea, hs, w1, b1r, w2, b2r)


def _combine_body(p0_ref, p1_ref, h_ref, root_ref, b_ref, out_ref):
    t = (p0_ref[...] + p1_ref[...]
         + jnp.dot(h_ref[...], root_ref[...], preferred_element_type=_f32)
         + b_ref[...])
    out_ref[...] = jnp.where(t > 0, t, jnp.exp(t) - 1.0)


def _combine(p0, p1, h, rootp, br):
    nb = _N // _NB
    return pl.pallas_call(
        _combine_body,
        grid=(nb,),
        in_specs=[
            pl.BlockSpec((_NB, _HP), lambda i: (i, 0)),
            pl.BlockSpec((_NB, _HP), lambda i: (i, 0)),
            pl.BlockSpec((_NB, _HP), lambda i: (i, 0)),
            pl.BlockSpec((_HP, _HP), lambda i: (0, 0)),
            pl.BlockSpec((1, _HP), lambda i: (0, 0)),
        ],
        out_specs=pl.BlockSpec((_NB, _HP), lambda i: (i, 0)),
        out_shape=jax.ShapeDtypeStruct((_N, _HP), _f32),
    )(p0, p1, h, rootp, br)


def _tail(h, x, n2s, s2g, w1p, b1r, w2, b2r, w3p, b3r):
    nb = _N // _TB
    n2s3 = n2s.reshape(nb, 1, _TB)
    s2g3 = s2g.reshape(1, 1, _S)
    pad = 128 - _H - _NF - 1

    def body(n2s_ref, s2g_ref, h_ref, x_ref, w1_ref, b1_ref, w2_ref, b2_ref,
             w3_ref, b3_ref, out_ref, acc):
        i = pl.program_id(0)
        ids = n2s_ref[0, 0, :]
        oh = (lax.broadcasted_iota(jnp.int32, (_S, _TB), 0)
              == ids[None, :]).astype(_f32)
        cat = jnp.concatenate(
            [h_ref[...][:, :_H], x_ref[...], jnp.ones((_TB, 1), _f32),
             jnp.zeros((_TB, pad), _f32)], axis=1)
        contrib = jnp.dot(oh, cat, preferred_element_type=_f32)

        @pl.when(i == 0)
        def _():
            acc[...] = contrib

        @pl.when(i > 0)
        def _():
            acc[...] = acc[...] + contrib

        @pl.when(i == nb - 1)
        def _():
            a = acc[...]
            cnt = a[:, 75:76]
            sm = a * (1.0 / jnp.maximum(cnt, 1.0))
            colmask = lax.broadcasted_iota(jnp.int32, (_S, 128), 1) == 75
            sm = jnp.where(colmask, 1.0, sm)
            gids = s2g_ref[0, 0, :]
            oh2 = (lax.broadcasted_iota(jnp.int32, (_G, _S), 0)
                   == gids[None, :]).astype(_f32)
            g = jnp.dot(oh2, sm, preferred_element_type=_f32)
            gm = g * (1.0 / jnp.maximum(g[:, 75:76], 1.0))
            r = jnp.dot(gm, w1_ref[...], preferred_element_type=_f32) + b1_ref[...]
            r = jnp.where(r > 0, r, jnp.exp(r) - 1.0)
            r = jnp.dot(r, w2_ref[...], preferred_element_type=_f32) + b2_ref[...]
            r = jnp.where(r > 0, r, jnp.exp(r) - 1.0)
            r = jnp.dot(r, w3_ref[...], preferred_element_type=_f32) + b3_ref[...]
            out_ref[...] = r[:, 0:1]

    return pl.pallas_call(
        body,
        grid=(nb,),
        in_specs=[
            pl.BlockSpec((1, 1, _TB), lambda i: (i, 0, 0)),
            pl.BlockSpec((1, 1, _S), lambda i: (0, 0, 0)),
            pl.BlockSpec((_TB, _HP), lambda i: (i, 0)),
            pl.BlockSpec((_TB, _NF), lambda i: (i, 0)),
            pl.BlockSpec((128, 32), lambda i: (0, 0)),
            pl.BlockSpec((1, 32), lambda i: (0, 0)),
            pl.BlockSpec((32, 16), lambda i: (0, 0)),
            pl.BlockSpec((1, 16), lambda i: (0, 0)),
            pl.BlockSpec((16, 128), lambda i: (0, 0)),
            pl.BlockSpec((1, 128), lambda i: (0, 0)),
        ],
        out_specs=pl.BlockSpec((_G, 1), lambda i: (0, 0)),
        out_shape=jax.ShapeDtypeStruct((_G, 1), _f32),
        scratch_shapes=[pltpu.VMEM((_S, 128), _f32)],
    )(n2s3, s2g3, h, x, w1p, b1r, w2, b2r, w3p, b3r)


def kernel(z, node_type, edge_index, edge_attr, x, node_to_subgraph,
           subgraph_to_graph, z_table, nt_table,
           w1_0, b1_0, w2_0, b2_0, root_0, bias_0,
           w1_1, b1_1, w2_1, b2_1, root_1, bias_1,
           w1_2, b1_2, w2_2, b2_2, root_2, bias_2,
           fc1_w, fc1_b, fc2_w, fc2_b, fc3_w, fc3_b):
    z = z.astype(jnp.int32)
    node_type = node_type.astype(jnp.int32)
    src = edge_index[0].astype(jnp.int32)
    dst = edge_index[1].astype(jnp.int32)
    ea = edge_attr.astype(_f32)

    zpad = jnp.zeros((_ZR, _HP - _H), _f32)
    ntpad = jnp.zeros((_NT, _HP - _H), _f32)
    ztp = jnp.concatenate([z_table, zpad], axis=1)
    nttp = jnp.concatenate([nt_table, ntpad], axis=1)
    h = _embed(z, node_type, ztp, nttp)
    zinit = jnp.zeros((_N // _NS, _HP), _f32)

    convs = [
        (w1_0, b1_0, w2_0, b2_0, root_0, bias_0),
        (w1_1, b1_1, w2_1, b2_1, root_1, bias_1),
        (w1_2, b1_2, w2_2, b2_2, root_2, bias_2),
    ]
    for (w1, b1, w2, b2, root, bias) in convs:
        hs = _gather_rows(h, src)
        msg = _dense(ea, hs, w1, b1.reshape(1, _KH), w2, b2.reshape(1, _HH))
        parts = _scatter_add(msg, dst, zinit)
        rootp = jnp.zeros((_HP, _HP), _f32).at[:_H, :_H].set(root)
        br = jnp.zeros((1, _HP), _f32).at[:, :_H].set(bias.reshape(1, _H))
        h = _combine(parts[:_N], parts[_N:], h, rootp, br)

    w1p = jnp.zeros((128, 32), _f32).at[:_H + _NF].set(fc1_w)
    w3p = jnp.zeros((16, 128), _f32).at[:, 0:1].set(fc3_w)
    b3r = jnp.broadcast_to(fc3_b.reshape(1, 1), (1, 128))
    out = _tail(h, x, node_to_subgraph.astype(jnp.int32),
                subgraph_to_graph.astype(jnp.int32),
                w1p, fc1_b.reshape(1, 32), fc2_w, fc2_b.reshape(1, 16),
                w3p, b3r)
    return out.reshape(-1)


# speedup vs baseline: 2.5548x; 2.5548x over previous
import functools

import jax
import jax.numpy as jnp
from jax import lax
from jax.experimental import pallas as pl
from jax.experimental.pallas import tpu as pltpu
from jax.experimental.pallas import tpu_sc as plsc

_N, _E, _S, _G = 10240, 40960, 640, 32
_NF, _EA, _H = 11, 5, 64
_HP = 128
_KH = 128
_HH = _H * _H
_ZR = 1000
_NT = 5
_NC, _NS = 2, 16
_NW = _NC * _NS
_EB = 256
_NB = 256
_TB = 512
_GCH = 128
_f32 = jnp.float32

_sc_mesh = plsc.VectorSubcoreMesh(
    core_axis_name="c", subcore_axis_name="s", num_cores=_NC, num_subcores=_NS)


def _emb_body(z_ref, nt_ref, zt_ref, ntt_ref, out_ref):
    z = z_ref[0, 0, :]
    nt = nt_ref[0, 0, :]
    ohz = (z[:, None] == lax.broadcasted_iota(jnp.int32, (_NB, _ZR), 1)).astype(_f32)
    ohn = (nt[:, None] == lax.broadcasted_iota(jnp.int32, (_NB, _NT), 1)).astype(_f32)
    out_ref[...] = (jnp.dot(ohz, zt_ref[...], preferred_element_type=_f32)
                    + jnp.dot(ohn, ntt_ref[...], preferred_element_type=_f32))


def _embed(z, node_type, ztp, nttp):
    nb = _N // _NB
    z3 = z.reshape(nb, 1, _NB)
    nt3 = node_type.reshape(nb, 1, _NB)
    return pl.pallas_call(
        _emb_body,
        grid=(nb,),
        in_specs=[
            pl.BlockSpec((1, 1, _NB), lambda i: (i, 0, 0)),
            pl.BlockSpec((1, 1, _NB), lambda i: (i, 0, 0)),
            pl.BlockSpec((_ZR, _HP), lambda i: (0, 0)),
            pl.BlockSpec((_NT, _HP), lambda i: (0, 0)),
        ],
        out_specs=pl.BlockSpec((_NB, _HP), lambda i: (i, 0)),
        out_shape=jax.ShapeDtypeStruct((_N, _HP), _f32),
    )(z3, nt3, ztp, nttp)


def _gather_rows(table, idx):
    per = _E // _NW
    nch = per // _GCH

    @functools.partial(
        pl.kernel,
        out_type=jax.ShapeDtypeStruct((_E, _HP), _f32),
        mesh=_sc_mesh,
        scratch_types=[
            pltpu.VMEM((_GCH,), jnp.int32),
            pltpu.VMEM((_GCH, _HP), _f32),
            pltpu.SemaphoreType.DMA,
        ],
    )
    def k(tab_hbm, idx_hbm, out_hbm, idx_v, rows_v, sem):
        wid = lax.axis_index("s") * _NC + lax.axis_index("c")
        base = wid * per
        for c in range(nch):
            off = base + c * _GCH
            pltpu.sync_copy(idx_hbm.at[pl.ds(off, _GCH)], idx_v)
            pltpu.async_copy(tab_hbm.at[idx_v], rows_v, sem).wait()
            pltpu.sync_copy(rows_v, out_hbm.at[pl.ds(off, _GCH)])

    return k(table, idx)


def _scatter_add(msg, dst, zinit):
    per = _E // _NW
    nch = per // _GCH
    rps = _N // _NS

    @functools.partial(
        pl.kernel,
        out_type=jax.ShapeDtypeStruct((_NC * _N, _HP), _f32),
        mesh=_sc_mesh,
        scratch_types=[
            pltpu.VMEM((_GCH,), jnp.int32),
            pltpu.VMEM((_GCH, _HP), _f32),
            pltpu.VMEM_SHARED((_N, _HP), _f32),
        ],
    )
    def k(msg_hbm, dst_hbm, z_hbm, out_hbm, idx_v, rows_v, agg_sh):
        cid = lax.axis_index("c")
        sid = lax.axis_index("s")
        wid = sid * _NC + cid
        pltpu.sync_copy(z_hbm, agg_sh.at[pl.ds(sid * rps, rps)])
        plsc.subcore_barrier()
        base = wid * per
        for c in range(nch):
            off = base + c * _GCH
            pltpu.sync_copy(dst_hbm.at[pl.ds(off, _GCH)], idx_v)
            pltpu.sync_copy(msg_hbm.at[pl.ds(off, _GCH)], rows_v)
            pltpu.sync_copy(rows_v, agg_sh.at[idx_v], add=True)
        plsc.subcore_barrier()
        pltpu.sync_copy(agg_sh.at[pl.ds(sid * rps, rps)],
                        out_hbm.at[pl.ds(cid * _N + sid * rps, rps)])

    return k(msg, dst, zinit)


def _dense_body(eat_ref, hs_ref, w1t_ref, b1t_ref, w2t_ref, b2mt_ref, out_ref):
    ft = jnp.maximum(
        jnp.dot(w1t_ref[...], eat_ref[...], preferred_element_type=_f32)
        + b1t_ref[...], 0.0)
    qt = jnp.dot(w2t_ref[...], ft, preferred_element_type=_f32)
    hst = hs_ref[...][:, :_H].T
    acc = None
    for i in range(_H):
        t = qt[i * _H:(i + 1) * _H, :] * hst[i:i + 1, :]
        acc = t if acc is None else acc + t
    msgt = acc + jnp.dot(b2mt_ref[...], hst, preferred_element_type=_f32)
    msg = msgt.T
    out_ref[...] = jnp.concatenate(
        [msg, jnp.zeros((_EB, _HP - _H), _f32)], axis=1)


def _dense(eat, hs, w1t, b1t, w2t, b2mt):
    nb = _E // _EB
    return pl.pallas_call(
        _dense_body,
        grid=(nb,),
        in_specs=[
            pl.BlockSpec((_EA, _EB), lambda i: (0, i)),
            pl.BlockSpec((_EB, _HP), lambda i: (i, 0)),
            pl.BlockSpec((_KH, _EA), lambda i: (0, 0)),
            pl.BlockSpec((_KH, 1), lambda i: (0, 0)),
            pl.BlockSpec((_HH, _KH), lambda i: (0, 0)),
            pl.BlockSpec((_H, _H), lambda i: (0, 0)),
        ],
        out_specs=pl.BlockSpec((_EB, _HP), lambda i: (i, 0)),
        out_shape=jax.ShapeDtypeStruct((_E, _HP), _f32),
    )(eat, hs, w1t, b1t, w2t, b2mt)


def _combine_body(p0_ref, p1_ref, h_ref, root_ref, b_ref, out_ref):
    t = (p0_ref[...] + p1_ref[...]
         + jnp.dot(h_ref[...], root_ref[...], preferred_element_type=_f32)
         + b_ref[...])
    out_ref[...] = jnp.where(t > 0, t, jnp.exp(t) - 1.0)


def _combine(p0, p1, h, rootp, br):
    nb = _N // _NB
    return pl.pallas_call(
        _combine_body,
        grid=(nb,),
        in_specs=[
            pl.BlockSpec((_NB, _HP), lambda i: (i, 0)),
            pl.BlockSpec((_NB, _HP), lambda i: (i, 0)),
            pl.BlockSpec((_NB, _HP), lambda i: (i, 0)),
            pl.BlockSpec((_HP, _HP), lambda i: (0, 0)),
            pl.BlockSpec((1, _HP), lambda i: (0, 0)),
        ],
        out_specs=pl.BlockSpec((_NB, _HP), lambda i: (i, 0)),
        out_shape=jax.ShapeDtypeStruct((_N, _HP), _f32),
    )(p0, p1, h, rootp, br)


def _tail(h, x, n2s, s2g, w1p, b1r, w2, b2r, w3p, b3r):
    nb = _N // _TB
    n2s3 = n2s.reshape(nb, 1, _TB)
    s2g3 = s2g.reshape(1, 1, _S)
    pad = 128 - _H - _NF - 1

    def body(n2s_ref, s2g_ref, h_ref, x_ref, w1_ref, b1_ref, w2_ref, b2_ref,
             w3_ref, b3_ref, out_ref, acc):
        i = pl.program_id(0)
        ids = n2s_ref[0, 0, :]
        oh = (lax.broadcasted_iota(jnp.int32, (_S, _TB), 0)
              == ids[None, :]).astype(_f32)
        cat = jnp.concatenate(
            [h_ref[...][:, :_H], x_ref[...], jnp.ones((_TB, 1), _f32),
             jnp.zeros((_TB, pad), _f32)], axis=1)
        contrib = jnp.dot(oh, cat, preferred_element_type=_f32)

        @pl.when(i == 0)
        def _():
            acc[...] = contrib

        @pl.when(i > 0)
        def _():
            acc[...] = acc[...] + contrib

        @pl.when(i == nb - 1)
        def _():
            a = acc[...]
            cnt = a[:, 75:76]
            sm = a * (1.0 / jnp.maximum(cnt, 1.0))
            colmask = lax.broadcasted_iota(jnp.int32, (_S, 128), 1) == 75
            sm = jnp.where(colmask, 1.0, sm)
            gids = s2g_ref[0, 0, :]
            oh2 = (lax.broadcasted_iota(jnp.int32, (_G, _S), 0)
                   == gids[None, :]).astype(_f32)
            g = jnp.dot(oh2, sm, preferred_element_type=_f32)
            gm = g * (1.0 / jnp.maximum(g[:, 75:76], 1.0))
            r = jnp.dot(gm, w1_ref[...], preferred_element_type=_f32) + b1_ref[...]
            r = jnp.where(r > 0, r, jnp.exp(r) - 1.0)
            r = jnp.dot(r, w2_ref[...], preferred_element_type=_f32) + b2_ref[...]
            r = jnp.where(r > 0, r, jnp.exp(r) - 1.0)
            r = jnp.dot(r, w3_ref[...], preferred_element_type=_f32) + b3_ref[...]
            out_ref[...] = r[:, 0:1]

    return pl.pallas_call(
        body,
        grid=(nb,),
        in_specs=[
            pl.BlockSpec((1, 1, _TB), lambda i: (i, 0, 0)),
            pl.BlockSpec((1, 1, _S), lambda i: (0, 0, 0)),
            pl.BlockSpec((_TB, _HP), lambda i: (i, 0)),
            pl.BlockSpec((_TB, _NF), lambda i: (i, 0)),
            pl.BlockSpec((128, 32), lambda i: (0, 0)),
            pl.BlockSpec((1, 32), lambda i: (0, 0)),
            pl.BlockSpec((32, 16), lambda i: (0, 0)),
            pl.BlockSpec((1, 16), lambda i: (0, 0)),
            pl.BlockSpec((16, 128), lambda i: (0, 0)),
            pl.BlockSpec((1, 128), lambda i: (0, 0)),
        ],
        out_specs=pl.BlockSpec((_G, 1), lambda i: (0, 0)),
        out_shape=jax.ShapeDtypeStruct((_G, 1), _f32),
        scratch_shapes=[pltpu.VMEM((_S, 128), _f32)],
    )(n2s3, s2g3, h, x, w1p, b1r, w2, b2r, w3p, b3r)


def kernel(z, node_type, edge_index, edge_attr, x, node_to_subgraph,
           subgraph_to_graph, z_table, nt_table,
           w1_0, b1_0, w2_0, b2_0, root_0, bias_0,
           w1_1, b1_1, w2_1, b2_1, root_1, bias_1,
           w1_2, b1_2, w2_2, b2_2, root_2, bias_2,
           fc1_w, fc1_b, fc2_w, fc2_b, fc3_w, fc3_b):
    z = z.astype(jnp.int32)
    node_type = node_type.astype(jnp.int32)
    src = edge_index[0].astype(jnp.int32)
    dst = edge_index[1].astype(jnp.int32)
    ea = edge_attr.astype(_f32)

    zpad = jnp.zeros((_ZR, _HP - _H), _f32)
    ntpad = jnp.zeros((_NT, _HP - _H), _f32)
    ztp = jnp.concatenate([z_table, zpad], axis=1)
    nttp = jnp.concatenate([nt_table, ntpad], axis=1)
    h = _embed(z, node_type, ztp, nttp)
    zinit = jnp.zeros((_N // _NS, _HP), _f32)

    convs = [
        (w1_0, b1_0, w2_0, b2_0, root_0, bias_0),
        (w1_1, b1_1, w2_1, b2_1, root_1, bias_1),
        (w1_2, b1_2, w2_2, b2_2, root_2, bias_2),
    ]
    eat = ea.T
    for (w1, b1, w2, b2, root, bias) in convs:
        hs = _gather_rows(h, src)
        msg = _dense(eat, hs, w1.T, b1.reshape(_KH, 1), w2.T,
                     b2.reshape(_H, _H).T)
        parts = _scatter_add(msg, dst, zinit)
        rootp = jnp.zeros((_HP, _HP), _f32).at[:_H, :_H].set(root)
        br = jnp.zeros((1, _HP), _f32).at[:, :_H].set(bias.reshape(1, _H))
        h = _combine(parts[:_N], parts[_N:], h, rootp, br)

    w1p = jnp.zeros((128, 32), _f32).at[:_H + _NF].set(fc1_w)
    w3p = jnp.zeros((16, 128), _f32).at[:, 0:1].set(fc3_w)
    b3r = jnp.broadcast_to(fc3_b.reshape(1, 1), (1, 128))
    out = _tail(h, x, node_to_subgraph.astype(jnp.int32),
                subgraph_to_graph.astype(jnp.int32),
                w1p, fc1_b.reshape(1, 32), fc2_w, fc2_b.reshape(1, 16),
                w3p, b3r)
    return out.reshape(-1)


# speedup vs baseline: 2.7385x; 1.0719x over previous
import functools

import jax
import jax.numpy as jnp
from jax import lax
from jax.experimental import pallas as pl
from jax.experimental.pallas import tpu as pltpu
from jax.experimental.pallas import tpu_sc as plsc

_N, _E, _S, _G = 10240, 40960, 640, 32
_NF, _EA, _H = 11, 5, 64
_HP = 128
_KH = 128
_HH = _H * _H
_ZR = 1000
_NT = 5
_NC, _NS = 2, 16
_NW = _NC * _NS
_EB = 256
_NB = 256
_TB = 512
_GCH = 128
_f32 = jnp.float32

_sc_mesh = plsc.VectorSubcoreMesh(
    core_axis_name="c", subcore_axis_name="s", num_cores=_NC, num_subcores=_NS)


def _emb_body(z_ref, nt_ref, zt_ref, ntt_ref, out_ref):
    z = z_ref[0, 0, :]
    nt = nt_ref[0, 0, :]
    ohz = (z[:, None] == lax.broadcasted_iota(jnp.int32, (_NB, _ZR), 1)).astype(_f32)
    ohn = (nt[:, None] == lax.broadcasted_iota(jnp.int32, (_NB, _NT), 1)).astype(_f32)
    out_ref[...] = (jnp.dot(ohz, zt_ref[...], preferred_element_type=_f32)
                    + jnp.dot(ohn, ntt_ref[...], preferred_element_type=_f32))


def _embed(z, node_type, ztp, nttp):
    nb = _N // _NB
    z3 = z.reshape(nb, 1, _NB)
    nt3 = node_type.reshape(nb, 1, _NB)
    return pl.pallas_call(
        _emb_body,
        grid=(nb,),
        in_specs=[
            pl.BlockSpec((1, 1, _NB), lambda i: (i, 0, 0)),
            pl.BlockSpec((1, 1, _NB), lambda i: (i, 0, 0)),
            pl.BlockSpec((_ZR, _HP), lambda i: (0, 0)),
            pl.BlockSpec((_NT, _HP), lambda i: (0, 0)),
        ],
        out_specs=pl.BlockSpec((_NB, _HP), lambda i: (i, 0)),
        out_shape=jax.ShapeDtypeStruct((_N, _HP), _f32),
    )(z3, nt3, ztp, nttp)


def _gather_rows(table, idx):
    per = _E // _NW
    nch = per // _GCH

    @functools.partial(
        pl.kernel,
        out_type=jax.ShapeDtypeStruct((_E, _HP), _f32),
        mesh=_sc_mesh,
        scratch_types=[
            pltpu.VMEM((2, _GCH), jnp.int32),
            pltpu.VMEM((2, _GCH, _HP), _f32),
            [pltpu.SemaphoreType.DMA] * 2,
            [pltpu.SemaphoreType.DMA] * 2,
            [pltpu.SemaphoreType.DMA] * 2,
        ],
    )
    def k(tab_hbm, idx_hbm, out_hbm, idx_v, rows_v, isems, gsems, ssems):
        wid = lax.axis_index("s") * _NC + lax.axis_index("c")
        base = wid * per

        def icopy(c):
            return pltpu.async_copy(
                idx_hbm.at[pl.ds(base + c * _GCH, _GCH)], idx_v.at[c % 2],
                isems[c % 2])

        def gcopy(c):
            return pltpu.async_copy(
                tab_hbm.at[idx_v.at[c % 2]], rows_v.at[c % 2], gsems[c % 2])

        def scopy(c):
            return pltpu.async_copy(
                rows_v.at[c % 2], out_hbm.at[pl.ds(base + c * _GCH, _GCH)],
                ssems[c % 2])

        ic = {0: icopy(0)}
        gc_, sc_ = {}, {}
        ic[0].wait()
        gc_[0] = gcopy(0)
        if nch > 1:
            ic[1] = icopy(1)
        for c in range(1, nch):
            ic[c].wait()
            if c >= 2:
                sc_[c - 2].wait()
            gc_[c] = gcopy(c)
            gc_[c - 1].wait()
            sc_[c - 1] = scopy(c - 1)
            if c + 1 < nch:
                ic[c + 1] = icopy(c + 1)
        gc_[nch - 1].wait()
        if nch >= 2:
            sc_[nch - 2].wait()
        sc_[nch - 1] = scopy(nch - 1)
        sc_[nch - 1].wait()

    return k(table, idx)


def _scatter_add(msg, dst, zinit):
    per = _E // _NW
    nch = per // _GCH
    rps = _N // _NS

    @functools.partial(
        pl.kernel,
        out_type=jax.ShapeDtypeStruct((_NC * _N, _HP), _f32),
        mesh=_sc_mesh,
        scratch_types=[
            pltpu.VMEM((2, _GCH), jnp.int32),
            pltpu.VMEM((2, _GCH, _HP), _f32),
            pltpu.VMEM_SHARED((_N, _HP), _f32),
            [pltpu.SemaphoreType.DMA] * 2,
            [pltpu.SemaphoreType.DMA] * 2,
            [pltpu.SemaphoreType.DMA] * 2,
        ],
    )
    def k(msg_hbm, dst_hbm, z_hbm, out_hbm, idx_v, rows_v, agg_sh,
          isems, msems, asems):
        cid = lax.axis_index("c")
        sid = lax.axis_index("s")
        wid = sid * _NC + cid
        base = wid * per

        def icopy(c):
            return pltpu.async_copy(
                dst_hbm.at[pl.ds(base + c * _GCH, _GCH)], idx_v.at[c % 2],
                isems[c % 2])

        def mcopy(c):
            return pltpu.async_copy(
                msg_hbm.at[pl.ds(base + c * _GCH, _GCH)], rows_v.at[c % 2],
                msems[c % 2])

        def acopy(c):
            return pltpu.async_copy(
                rows_v.at[c % 2], agg_sh.at[idx_v.at[c % 2]], asems[c % 2],
                add=True)

        ic = {0: icopy(0)}
        mc = {0: mcopy(0)}
        pltpu.sync_copy(z_hbm, agg_sh.at[pl.ds(sid * rps, rps)])
        plsc.subcore_barrier()
        ac = {}
        for c in range(nch):
            ic[c].wait()
            mc[c].wait()
            if c >= 1:
                ac[c - 1].wait()
            if c + 1 < nch:
                ic[c + 1] = icopy(c + 1)
                mc[c + 1] = mcopy(c + 1)
            ac[c] = acopy(c)
        ac[nch - 1].wait()
        plsc.subcore_barrier()
        pltpu.sync_copy(agg_sh.at[pl.ds(sid * rps, rps)],
                        out_hbm.at[pl.ds(cid * _N + sid * rps, rps)])

    return k(msg, dst, zinit)


def _dense_body(eat_ref, hs_ref, w1t_ref, b1t_ref, w2t_ref, b2mt_ref, out_ref):
    ft = jnp.maximum(
        jnp.dot(w1t_ref[...], eat_ref[...], preferred_element_type=_f32)
        + b1t_ref[...], 0.0)
    qt = jnp.dot(w2t_ref[...], ft, preferred_element_type=_f32)
    hst = hs_ref[...][:, :_H].T
    acc = None
    for i in range(_H):
        t = qt[i * _H:(i + 1) * _H, :] * hst[i:i + 1, :]
        acc = t if acc is None else acc + t
    msgt = acc + jnp.dot(b2mt_ref[...], hst, preferred_element_type=_f32)
    msg = msgt.T
    out_ref[...] = jnp.concatenate(
        [msg, jnp.zeros((_EB, _HP - _H), _f32)], axis=1)


def _dense(eat, hs, w1t, b1t, w2t, b2mt):
    nb = _E // _EB
    return pl.pallas_call(
        _dense_body,
        grid=(nb,),
        in_specs=[
            pl.BlockSpec((_EA, _EB), lambda i: (0, i)),
            pl.BlockSpec((_EB, _HP), lambda i: (i, 0)),
            pl.BlockSpec((_KH, _EA), lambda i: (0, 0)),
            pl.BlockSpec((_KH, 1), lambda i: (0, 0)),
            pl.BlockSpec((_HH, _KH), lambda i: (0, 0)),
            pl.BlockSpec((_H, _H), lambda i: (0, 0)),
        ],
        out_specs=pl.BlockSpec((_EB, _HP), lambda i: (i, 0)),
        out_shape=jax.ShapeDtypeStruct((_E, _HP), _f32),
    )(eat, hs, w1t, b1t, w2t, b2mt)


def _combine_body(p0_ref, p1_ref, h_ref, root_ref, b_ref, out_ref):
    t = (p0_ref[...] + p1_ref[...]
         + jnp.dot(h_ref[...], root_ref[...], preferred_element_type=_f32)
         + b_ref[...])
    out_ref[...] = jnp.where(t > 0, t, jnp.exp(t) - 1.0)


def _combine(p0, p1, h, rootp, br):
    nb = _N // _NB
    return pl.pallas_call(
        _combine_body,
        grid=(nb,),
        in_specs=[
            pl.BlockSpec((_NB, _HP), lambda i: (i, 0)),
            pl.BlockSpec((_NB, _HP), lambda i: (i, 0)),
            pl.BlockSpec((_NB, _HP), lambda i: (i, 0)),
            pl.BlockSpec((_HP, _HP), lambda i: (0, 0)),
            pl.BlockSpec((1, _HP), lambda i: (0, 0)),
        ],
        out_specs=pl.BlockSpec((_NB, _HP), lambda i: (i, 0)),
        out_shape=jax.ShapeDtypeStruct((_N, _HP), _f32),
    )(p0, p1, h, rootp, br)


def _tail(h, x, n2s, s2g, w1p, b1r, w2, b2r, w3p, b3r):
    nb = _N // _TB
    n2s3 = n2s.reshape(nb, 1, _TB)
    s2g3 = s2g.reshape(1, 1, _S)
    pad = 128 - _H - _NF - 1

    def body(n2s_ref, s2g_ref, h_ref, x_ref, w1_ref, b1_ref, w2_ref, b2_ref,
             w3_ref, b3_ref, out_ref, acc):
        i = pl.program_id(0)
        ids = n2s_ref[0, 0, :]
        oh = (lax.broadcasted_iota(jnp.int32, (_S, _TB), 0)
              == ids[None, :]).astype(_f32)
        cat = jnp.concatenate(
            [h_ref[...][:, :_H], x_ref[...], jnp.ones((_TB, 1), _f32),
             jnp.zeros((_TB, pad), _f32)], axis=1)
        contrib = jnp.dot(oh, cat, preferred_element_type=_f32)

        @pl.when(i == 0)
        def _():
            acc[...] = contrib

        @pl.when(i > 0)
        def _():
            acc[...] = acc[...] + contrib

        @pl.when(i == nb - 1)
        def _():
            a = acc[...]
            cnt = a[:, 75:76]
            sm = a * (1.0 / jnp.maximum(cnt, 1.0))
            colmask = lax.broadcasted_iota(jnp.int32, (_S, 128), 1) == 75
            sm = jnp.where(colmask, 1.0, sm)
            gids = s2g_ref[0, 0, :]
            oh2 = (lax.broadcasted_iota(jnp.int32, (_G, _S), 0)
                   == gids[None, :]).astype(_f32)
            g = jnp.dot(oh2, sm, preferred_element_type=_f32)
            gm = g * (1.0 / jnp.maximum(g[:, 75:76], 1.0))
            r = jnp.dot(gm, w1_ref[...], preferred_element_type=_f32) + b1_ref[...]
            r = jnp.where(r > 0, r, jnp.exp(r) - 1.0)
            r = jnp.dot(r, w2_ref[...], preferred_element_type=_f32) + b2_ref[...]
            r = jnp.where(r > 0, r, jnp.exp(r) - 1.0)
            r = jnp.dot(r, w3_ref[...], preferred_element_type=_f32) + b3_ref[...]
            out_ref[...] = r[:, 0:1]

    return pl.pallas_call(
        body,
        grid=(nb,),
        in_specs=[
            pl.BlockSpec((1, 1, _TB), lambda i: (i, 0, 0)),
            pl.BlockSpec((1, 1, _S), lambda i: (0, 0, 0)),
            pl.BlockSpec((_TB, _HP), lambda i: (i, 0)),
            pl.BlockSpec((_TB, _NF), lambda i: (i, 0)),
            pl.BlockSpec((128, 32), lambda i: (0, 0)),
            pl.BlockSpec((1, 32), lambda i: (0, 0)),
            pl.BlockSpec((32, 16), lambda i: (0, 0)),
            pl.BlockSpec((1, 16), lambda i: (0, 0)),
            pl.BlockSpec((16, 128), lambda i: (0, 0)),
            pl.BlockSpec((1, 128), lambda i: (0, 0)),
        ],
        out_specs=pl.BlockSpec((_G, 1), lambda i: (0, 0)),
        out_shape=jax.ShapeDtypeStruct((_G, 1), _f32),
        scratch_shapes=[pltpu.VMEM((_S, 128), _f32)],
    )(n2s3, s2g3, h, x, w1p, b1r, w2, b2r, w3p, b3r)


def kernel(z, node_type, edge_index, edge_attr, x, node_to_subgraph,
           subgraph_to_graph, z_table, nt_table,
           w1_0, b1_0, w2_0, b2_0, root_0, bias_0,
           w1_1, b1_1, w2_1, b2_1, root_1, bias_1,
           w1_2, b1_2, w2_2, b2_2, root_2, bias_2,
           fc1_w, fc1_b, fc2_w, fc2_b, fc3_w, fc3_b):
    z = z.astype(jnp.int32)
    node_type = node_type.astype(jnp.int32)
    src = edge_index[0].astype(jnp.int32)
    dst = edge_index[1].astype(jnp.int32)
    ea = edge_attr.astype(_f32)

    zpad = jnp.zeros((_ZR, _HP - _H), _f32)
    ntpad = jnp.zeros((_NT, _HP - _H), _f32)
    ztp = jnp.concatenate([z_table, zpad], axis=1)
    nttp = jnp.concatenate([nt_table, ntpad], axis=1)
    h = _embed(z, node_type, ztp, nttp)
    zinit = jnp.zeros((_N // _NS, _HP), _f32)

    convs = [
        (w1_0, b1_0, w2_0, b2_0, root_0, bias_0),
        (w1_1, b1_1, w2_1, b2_1, root_1, bias_1),
        (w1_2, b1_2, w2_2, b2_2, root_2, bias_2),
    ]
    eat = ea.T
    for (w1, b1, w2, b2, root, bias) in convs:
        hs = _gather_rows(h, src)
        msg = _dense(eat, hs, w1.T, b1.reshape(_KH, 1), w2.T,
                     b2.reshape(_H, _H).T)
        parts = _scatter_add(msg, dst, zinit)
        rootp = jnp.zeros((_HP, _HP), _f32).at[:_H, :_H].set(root)
        br = jnp.zeros((1, _HP), _f32).at[:, :_H].set(bias.reshape(1, _H))
        h = _combine(parts[:_N], parts[_N:], h, rootp, br)

    w1p = jnp.zeros((128, 32), _f32).at[:_H + _NF].set(fc1_w)
    w3p = jnp.zeros((16, 128), _f32).at[:, 0:1].set(fc3_w)
    b3r = jnp.broadcast_to(fc3_b.reshape(1, 1), (1, 128))
    out = _tail(h, x, node_to_subgraph.astype(jnp.int32),
                subgraph_to_graph.astype(jnp.int32),
                w1p, fc1_b.reshape(1, 32), fc2_w, fc2_b.reshape(1, 16),
                w3p, b3r)
    return out.reshape(-1)


# speedup vs baseline: 3.2770x; 1.1966x over previous
import functools

import jax
import jax.numpy as jnp
from jax import lax
from jax.experimental import pallas as pl
from jax.experimental.pallas import tpu as pltpu
from jax.experimental.pallas import tpu_sc as plsc

_N, _E, _S, _G = 10240, 40960, 640, 32
_NF, _EA, _H = 11, 5, 64
_HP = 128
_KH = 128
_HH = _H * _H
_ZR = 1000
_NT = 5
_NC, _NS = 2, 16
_NW = _NC * _NS
_EB = 512
_NB = 256
_TB = 512
_GCH = 128
_f32 = jnp.float32

_sc_mesh = plsc.VectorSubcoreMesh(
    core_axis_name="c", subcore_axis_name="s", num_cores=_NC, num_subcores=_NS)


def _emb_body(z_ref, nt_ref, zt_ref, ntt_ref, out_ref):
    z = z_ref[0, 0, :]
    nt = nt_ref[0, 0, :]
    ohz = (z[:, None] == lax.broadcasted_iota(jnp.int32, (_NB, _ZR), 1)).astype(_f32)
    ohn = (nt[:, None] == lax.broadcasted_iota(jnp.int32, (_NB, _NT), 1)).astype(_f32)
    out_ref[...] = (jnp.dot(ohz, zt_ref[...], preferred_element_type=_f32)
                    + jnp.dot(ohn, ntt_ref[...], preferred_element_type=_f32))


def _embed(z, node_type, ztp, nttp):
    nb = _N // _NB
    z3 = z.reshape(nb, 1, _NB)
    nt3 = node_type.reshape(nb, 1, _NB)
    return pl.pallas_call(
        _emb_body,
        grid=(nb,),
        in_specs=[
            pl.BlockSpec((1, 1, _NB), lambda i: (i, 0, 0)),
            pl.BlockSpec((1, 1, _NB), lambda i: (i, 0, 0)),
            pl.BlockSpec((_ZR, _HP), lambda i: (0, 0)),
            pl.BlockSpec((_NT, _HP), lambda i: (0, 0)),
        ],
        out_specs=pl.BlockSpec((_NB, _HP), lambda i: (i, 0)),
        out_shape=jax.ShapeDtypeStruct((_N, _HP), _f32),
    )(z3, nt3, ztp, nttp)


def _gather_rows(table, idx):
    per = _E // _NW
    nch = per // _GCH

    @functools.partial(
        pl.kernel,
        out_type=jax.ShapeDtypeStruct((_E, _HP), _f32),
        mesh=_sc_mesh,
        scratch_types=[
            pltpu.VMEM((2, _GCH), jnp.int32),
            pltpu.VMEM((2, _GCH, _HP), _f32),
            [pltpu.SemaphoreType.DMA] * 2,
            [pltpu.SemaphoreType.DMA] * 2,
            [pltpu.SemaphoreType.DMA] * 2,
        ],
    )
    def k(tab_hbm, idx_hbm, out_hbm, idx_v, rows_v, isems, gsems, ssems):
        wid = lax.axis_index("s") * _NC + lax.axis_index("c")
        base = wid * per

        def icopy(c):
            return pltpu.async_copy(
                idx_hbm.at[pl.ds(base + c * _GCH, _GCH)], idx_v.at[c % 2],
                isems[c % 2])

        def gcopy(c):
            return pltpu.async_copy(
                tab_hbm.at[idx_v.at[c % 2]], rows_v.at[c % 2], gsems[c % 2])

        def scopy(c):
            return pltpu.async_copy(
                rows_v.at[c % 2], out_hbm.at[pl.ds(base + c * _GCH, _GCH)],
                ssems[c % 2])

        ic = {0: icopy(0)}
        gc_, sc_ = {}, {}
        ic[0].wait()
        gc_[0] = gcopy(0)
        if nch > 1:
            ic[1] = icopy(1)
        for c in range(1, nch):
            ic[c].wait()
            if c >= 2:
                sc_[c - 2].wait()
            gc_[c] = gcopy(c)
            gc_[c - 1].wait()
            sc_[c - 1] = scopy(c - 1)
            if c + 1 < nch:
                ic[c + 1] = icopy(c + 1)
        gc_[nch - 1].wait()
        if nch >= 2:
            sc_[nch - 2].wait()
        sc_[nch - 1] = scopy(nch - 1)
        sc_[nch - 1].wait()

    return k(table, idx)


def _scatter_add(msg, dst, zinit):
    per = _E // _NW
    nch = per // _GCH
    rps = _N // _NS

    @functools.partial(
        pl.kernel,
        out_type=jax.ShapeDtypeStruct((_NC * _N, _HP), _f32),
        mesh=_sc_mesh,
        scratch_types=[
            pltpu.VMEM((2, _GCH), jnp.int32),
            pltpu.VMEM((2, _GCH, _HP), _f32),
            pltpu.VMEM_SHARED((_N, _HP), _f32),
            [pltpu.SemaphoreType.DMA] * 2,
            [pltpu.SemaphoreType.DMA] * 2,
            [pltpu.SemaphoreType.DMA] * 2,
        ],
    )
    def k(msg_hbm, dst_hbm, z_hbm, out_hbm, idx_v, rows_v, agg_sh,
          isems, msems, asems):
        cid = lax.axis_index("c")
        sid = lax.axis_index("s")
        wid = sid * _NC + cid
        base = wid * per

        def icopy(c):
            return pltpu.async_copy(
                dst_hbm.at[pl.ds(base + c * _GCH, _GCH)], idx_v.at[c % 2],
                isems[c % 2])

        def mcopy(c):
            return pltpu.async_copy(
                msg_hbm.at[pl.ds(base + c * _GCH, _GCH)], rows_v.at[c % 2],
                msems[c % 2])

        def acopy(c):
            return pltpu.async_copy(
                rows_v.at[c % 2], agg_sh.at[idx_v.at[c % 2]], asems[c % 2],
                add=True)

        ic = {0: icopy(0)}
        mc = {0: mcopy(0)}
        pltpu.sync_copy(z_hbm, agg_sh.at[pl.ds(sid * rps, rps)])
        plsc.subcore_barrier()
        ac = {}
        for c in range(nch):
            ic[c].wait()
            mc[c].wait()
            if c >= 1:
                ac[c - 1].wait()
            if c + 1 < nch:
                ic[c + 1] = icopy(c + 1)
                mc[c + 1] = mcopy(c + 1)
            ac[c] = acopy(c)
        ac[nch - 1].wait()
        plsc.subcore_barrier()
        pltpu.sync_copy(agg_sh.at[pl.ds(sid * rps, rps)],
                        out_hbm.at[pl.ds(cid * _N + sid * rps, rps)])

    return k(msg, dst, zinit)


def _dense_body(eat_ref, hs_ref, w1t_ref, b1t_ref, w2t_ref, b2mt_ref, out_ref):
    ft = jnp.maximum(
        jnp.dot(w1t_ref[...], eat_ref[...], preferred_element_type=_f32)
        + b1t_ref[...], 0.0)
    qt = jnp.dot(w2t_ref[...], ft, preferred_element_type=_f32)
    hst = hs_ref[...][:, :_H].T
    acc = None
    for i in range(_H):
        t = qt[i * _H:(i + 1) * _H, :] * hst[i:i + 1, :]
        acc = t if acc is None else acc + t
    msgt = acc + jnp.dot(b2mt_ref[...], hst, preferred_element_type=_f32)
    msg = msgt.T
    out_ref[...] = jnp.concatenate(
        [msg, jnp.zeros((_EB, _HP - _H), _f32)], axis=1)


def _dense(eat, hs, w1t, b1t, w2t, b2mt):
    nb = _E // _EB
    return pl.pallas_call(
        _dense_body,
        grid=(nb,),
        in_specs=[
            pl.BlockSpec((_EA, _EB), lambda i: (0, i)),
            pl.BlockSpec((_EB, _HP), lambda i: (i, 0)),
            pl.BlockSpec((_KH, _EA), lambda i: (0, 0)),
            pl.BlockSpec((_KH, 1), lambda i: (0, 0)),
            pl.BlockSpec((_HH, _KH), lambda i: (0, 0)),
            pl.BlockSpec((_H, _H), lambda i: (0, 0)),
        ],
        out_specs=pl.BlockSpec((_EB, _HP), lambda i: (i, 0)),
        out_shape=jax.ShapeDtypeStruct((_E, _HP), _f32),
    )(eat, hs, w1t, b1t, w2t, b2mt)


def _combine_body(p0_ref, p1_ref, h_ref, root_ref, b_ref, out_ref):
    t = (p0_ref[...] + p1_ref[...]
         + jnp.dot(h_ref[...], root_ref[...], preferred_element_type=_f32)
         + b_ref[...])
    out_ref[...] = jnp.where(t > 0, t, jnp.exp(t) - 1.0)


def _combine(p0, p1, h, rootp, br):
    nb = _N // _NB
    return pl.pallas_call(
        _combine_body,
        grid=(nb,),
        in_specs=[
            pl.BlockSpec((_NB, _HP), lambda i: (i, 0)),
            pl.BlockSpec((_NB, _HP), lambda i: (i, 0)),
            pl.BlockSpec((_NB, _HP), lambda i: (i, 0)),
            pl.BlockSpec((_HP, _HP), lambda i: (0, 0)),
            pl.BlockSpec((1, _HP), lambda i: (0, 0)),
        ],
        out_specs=pl.BlockSpec((_NB, _HP), lambda i: (i, 0)),
        out_shape=jax.ShapeDtypeStruct((_N, _HP), _f32),
    )(p0, p1, h, rootp, br)


def _tail(h, x, n2s, s2g, w1p, b1r, w2, b2r, w3p, b3r):
    nb = _N // _TB
    n2s3 = n2s.reshape(nb, 1, _TB)
    s2g3 = s2g.reshape(1, 1, _S)
    pad = 128 - _H - _NF - 1

    def body(n2s_ref, s2g_ref, h_ref, x_ref, w1_ref, b1_ref, w2_ref, b2_ref,
             w3_ref, b3_ref, out_ref, acc):
        i = pl.program_id(0)
        ids = n2s_ref[0, 0, :]
        oh = (lax.broadcasted_iota(jnp.int32, (_S, _TB), 0)
              == ids[None, :]).astype(_f32)
        cat = jnp.concatenate(
            [h_ref[...][:, :_H], x_ref[...], jnp.ones((_TB, 1), _f32),
             jnp.zeros((_TB, pad), _f32)], axis=1)
        contrib = jnp.dot(oh, cat, preferred_element_type=_f32)

        @pl.when(i == 0)
        def _():
            acc[...] = contrib

        @pl.when(i > 0)
        def _():
            acc[...] = acc[...] + contrib

        @pl.when(i == nb - 1)
        def _():
            a = acc[...]
            cnt = a[:, 75:76]
            sm = a * (1.0 / jnp.maximum(cnt, 1.0))
            colmask = lax.broadcasted_iota(jnp.int32, (_S, 128), 1) == 75
            sm = jnp.where(colmask, 1.0, sm)
            gids = s2g_ref[0, 0, :]
            oh2 = (lax.broadcasted_iota(jnp.int32, (_G, _S), 0)
                   == gids[None, :]).astype(_f32)
            g = jnp.dot(oh2, sm, preferred_element_type=_f32)
            gm = g * (1.0 / jnp.maximum(g[:, 75:76], 1.0))
            r = jnp.dot(gm, w1_ref[...], preferred_element_type=_f32) + b1_ref[...]
            r = jnp.where(r > 0, r, jnp.exp(r) - 1.0)
            r = jnp.dot(r, w2_ref[...], preferred_element_type=_f32) + b2_ref[...]
            r = jnp.where(r > 0, r, jnp.exp(r) - 1.0)
            r = jnp.dot(r, w3_ref[...], preferred_element_type=_f32) + b3_ref[...]
            out_ref[...] = r[:, 0:1]

    return pl.pallas_call(
        body,
        grid=(nb,),
        in_specs=[
            pl.BlockSpec((1, 1, _TB), lambda i: (i, 0, 0)),
            pl.BlockSpec((1, 1, _S), lambda i: (0, 0, 0)),
            pl.BlockSpec((_TB, _HP), lambda i: (i, 0)),
            pl.BlockSpec((_TB, _NF), lambda i: (i, 0)),
            pl.BlockSpec((128, 32), lambda i: (0, 0)),
            pl.BlockSpec((1, 32), lambda i: (0, 0)),
            pl.BlockSpec((32, 16), lambda i: (0, 0)),
            pl.BlockSpec((1, 16), lambda i: (0, 0)),
            pl.BlockSpec((16, 128), lambda i: (0, 0)),
            pl.BlockSpec((1, 128), lambda i: (0, 0)),
        ],
        out_specs=pl.BlockSpec((_G, 1), lambda i: (0, 0)),
        out_shape=jax.ShapeDtypeStruct((_G, 1), _f32),
        scratch_shapes=[pltpu.VMEM((_S, 128), _f32)],
    )(n2s3, s2g3, h, x, w1p, b1r, w2, b2r, w3p, b3r)


def kernel(z, node_type, edge_index, edge_attr, x, node_to_subgraph,
           subgraph_to_graph, z_table, nt_table,
           w1_0, b1_0, w2_0, b2_0, root_0, bias_0,
           w1_1, b1_1, w2_1, b2_1, root_1, bias_1,
           w1_2, b1_2, w2_2, b2_2, root_2, bias_2,
           fc1_w, fc1_b, fc2_w, fc2_b, fc3_w, fc3_b):
    z = z.astype(jnp.int32)
    node_type = node_type.astype(jnp.int32)
    src = edge_index[0].astype(jnp.int32)
    dst = edge_index[1].astype(jnp.int32)
    ea = edge_attr.astype(_f32)

    zpad = jnp.zeros((_ZR, _HP - _H), _f32)
    ntpad = jnp.zeros((_NT, _HP - _H), _f32)
    ztp = jnp.concatenate([z_table, zpad], axis=1)
    nttp = jnp.concatenate([nt_table, ntpad], axis=1)
    h = _embed(z, node_type, ztp, nttp)
    zinit = jnp.zeros((_N // _NS, _HP), _f32)

    convs = [
        (w1_0, b1_0, w2_0, b2_0, root_0, bias_0),
        (w1_1, b1_1, w2_1, b2_1, root_1, bias_1),
        (w1_2, b1_2, w2_2, b2_2, root_2, bias_2),
    ]
    eat = ea.T
    for (w1, b1, w2, b2, root, bias) in convs:
        hs = _gather_rows(h, src)
        msg = _dense(eat, hs, w1.T, b1.reshape(_KH, 1), w2.T,
                     b2.reshape(_H, _H).T)
        parts = _scatter_add(msg, dst, zinit)
        rootp = jnp.zeros((_HP, _HP), _f32).at[:_H, :_H].set(root)
        br = jnp.zeros((1, _HP), _f32).at[:, :_H].set(bias.reshape(1, _H))
        h = _combine(parts[:_N], parts[_N:], h, rootp, br)

    w1p = jnp.zeros((128, 32), _f32).at[:_H + _NF].set(fc1_w)
    w3p = jnp.zeros((16, 128), _f32).at[:, 0:1].set(fc3_w)
    b3r = jnp.broadcast_to(fc3_b.reshape(1, 1), (1, 128))
    out = _tail(h, x, node_to_subgraph.astype(jnp.int32),
                subgraph_to_graph.astype(jnp.int32),
                w1p, fc1_b.reshape(1, 32), fc2_w, fc2_b.reshape(1, 16),
                w3p, b3r)
    return out.reshape(-1)
